# baseline (speedup 1.0000x reference)
import functools

import jax
import jax.numpy as jnp
from jax import lax
from jax.experimental import pallas as pl
from jax.experimental.pallas import tpu as pltpu
from jax.experimental.pallas import tpu_sc as plsc

BATCH = 4096
SEQ = 200
EMBED = 64
N_ROWS = BATCH * SEQ
NUM_WORKERS = 32
ROWS_PER_WORKER = N_ROWS // NUM_WORKERS
CHUNK = 100
NUM_CHUNKS = ROWS_PER_WORKER // CHUNK
NBUF = 4

_MESH = plsc.VectorSubcoreMesh(
    core_axis_name="c", subcore_axis_name="s", num_cores=2, num_subcores=16
)


@functools.partial(
    pl.kernel,
    out_type=jax.ShapeDtypeStruct((BATCH, SEQ, EMBED), jnp.float32),
    mesh=_MESH,
    compiler_params=pltpu.CompilerParams(use_tc_tiling_on_sc=False),
    scratch_types=[
        pltpu.VMEM((NUM_CHUNKS, CHUNK), jnp.int32),
        pltpu.VMEM((NUM_CHUNKS, CHUNK), jnp.int32),
    ]
    + [pltpu.VMEM((CHUNK, EMBED), jnp.float32) for _ in range(NBUF)]
    + [pltpu.SemaphoreType.DMA] * (3 * NBUF),
)
def _embed_sum(notes_hbm, times_hbm, tok_hbm, pos_hbm, out_hbm,
               idx_n, idx_t, *bufs_and_sems):
    bufs = bufs_and_sems[:NBUF]
    sem_a = bufs_and_sems[NBUF:2 * NBUF]
    sem_b = bufs_and_sems[2 * NBUF:3 * NBUF]
    sem_c = bufs_and_sems[3 * NBUF:4 * NBUF]

    w = lax.axis_index("s") * 2 + lax.axis_index("c")
    pltpu.sync_copy(notes_hbm.at[w], idx_n)
    pltpu.sync_copy(times_hbm.at[w], idx_t)
    batch_base = w * (BATCH // NUM_WORKERS)

    def step(g):
        for b in range(NBUF):
            c = g + b
            b1 = (b - 1) % NBUF
            b2 = (b - 2) % NBUF

            @pl.when(jnp.logical_and(c >= NBUF, c - NBUF < NUM_CHUNKS))
            def _():
                pltpu.make_async_copy(
                    bufs[b], out_hbm.at[batch_base, pl.ds(0, CHUNK)], sem_c[b]
                ).wait()

            @pl.when(c < NUM_CHUNKS)
            def _():
                pltpu.async_copy(tok_hbm.at[idx_n.at[c]], bufs[b], sem_a[b])

            c1 = c - 1
            @pl.when(jnp.logical_and(c1 >= 0, c1 < NUM_CHUNKS))
            def _():
                pltpu.make_async_copy(
                    tok_hbm.at[idx_n.at[c1]], bufs[b1], sem_a[b1]
                ).wait()
                pltpu.async_copy(
                    pos_hbm.at[idx_t.at[c1]], bufs[b1], sem_b[b1], add=True
                )

            c2 = c - 2
            @pl.when(jnp.logical_and(c2 >= 0, c2 < NUM_CHUNKS))
            def _():
                pltpu.make_async_copy(
                    pos_hbm.at[idx_t.at[c2]], bufs[b2], sem_b[b2]
                ).wait()
                pltpu.async_copy(
                    bufs[b2],
                    out_hbm.at[batch_base + c2 // 2, pl.ds((c2 % 2) * CHUNK, CHUNK)],
                    sem_c[b2],
                )

    pl.loop(0, NUM_CHUNKS + NBUF, step=NBUF)(step)


def kernel(x, token_emb, pos_emb):
    notes = x[:, 0, :].astype(jnp.int32).reshape(NUM_WORKERS, NUM_CHUNKS, CHUNK)
    times = x[:, 1, :].astype(jnp.int32).reshape(NUM_WORKERS, NUM_CHUNKS, CHUNK)
    return _embed_sum(notes, times, token_emb, pos_emb)


# speedup vs baseline: 1.2630x; 1.2630x over previous
import functools

import jax
import jax.numpy as jnp
from jax import lax
from jax.experimental import pallas as pl
from jax.experimental.pallas import tpu as pltpu
from jax.experimental.pallas import tpu_sc as plsc
from jax.experimental.layout import Format, Layout, with_layout_constraint

BATCH = 4096
SEQ = 200
EMBED = 64
N_ROWS = BATCH * SEQ
NUM_WORKERS = 32
ROWS_PER_WORKER = N_ROWS // NUM_WORKERS
CHUNK = 100
NUM_CHUNKS = ROWS_PER_WORKER // CHUNK
NBUF = 4

_MESH = plsc.VectorSubcoreMesh(
    core_axis_name="c", subcore_axis_name="s", num_cores=2, num_subcores=16
)


@functools.partial(
    pl.kernel,
    out_type=jax.ShapeDtypeStruct((BATCH, SEQ, EMBED), jnp.float32),
    mesh=_MESH,
    compiler_params=pltpu.CompilerParams(use_tc_tiling_on_sc=False),
    scratch_types=[
        pltpu.VMEM((NUM_CHUNKS, CHUNK), jnp.int32),
        pltpu.VMEM((NUM_CHUNKS, CHUNK), jnp.int32),
    ]
    + [pltpu.VMEM((CHUNK, EMBED), jnp.float32) for _ in range(NBUF)]
    + [pltpu.SemaphoreType.DMA] * (3 * NBUF),
)
def _embed_sum(notes_hbm, times_hbm, tok_hbm, pos_hbm, out_hbm,
               idx_n, idx_t, *bufs_and_sems):
    bufs = bufs_and_sems[:NBUF]
    sem_a = bufs_and_sems[NBUF:2 * NBUF]
    sem_b = bufs_and_sems[2 * NBUF:3 * NBUF]
    sem_c = bufs_and_sems[3 * NBUF:4 * NBUF]

    w = lax.axis_index("s") * 2 + lax.axis_index("c")
    pltpu.sync_copy(notes_hbm.at[w], idx_n)
    pltpu.sync_copy(times_hbm.at[w], idx_t)
    batch_base = w * (BATCH // NUM_WORKERS)

    def step(g):
        for b in range(NBUF):
            c = g + b
            b1 = (b - 1) % NBUF
            b2 = (b - 2) % NBUF

            @pl.when(jnp.logical_and(c >= NBUF, c - NBUF < NUM_CHUNKS))
            def _():
                pltpu.make_async_copy(
                    bufs[b], out_hbm.at[batch_base, pl.ds(0, CHUNK)], sem_c[b]
                ).wait()

            @pl.when(c < NUM_CHUNKS)
            def _():
                pltpu.async_copy(tok_hbm.at[idx_n.at[c]], bufs[b], sem_a[b])

            c1 = c - 1
            @pl.when(jnp.logical_and(c1 >= 0, c1 < NUM_CHUNKS))
            def _():
                pltpu.make_async_copy(
                    tok_hbm.at[idx_n.at[c1]], bufs[b1], sem_a[b1]
                ).wait()
                pltpu.async_copy(
                    pos_hbm.at[idx_t.at[c1]], bufs[b1], sem_b[b1], add=True
                )

            c2 = c - 2
            @pl.when(jnp.logical_and(c2 >= 0, c2 < NUM_CHUNKS))
            def _():
                pltpu.make_async_copy(
                    pos_hbm.at[idx_t.at[c2]], bufs[b2], sem_b[b2]
                ).wait()
                pltpu.async_copy(
                    bufs[b2],
                    out_hbm.at[batch_base + c2 // 2, pl.ds((c2 % 2) * CHUNK, CHUNK)],
                    sem_c[b2],
                )

    pl.loop(0, NUM_CHUNKS + NBUF, step=NBUF)(step)


def kernel(x, token_emb, pos_emb):
    notes = x[:, 0, :].astype(jnp.int32).reshape(NUM_WORKERS, NUM_CHUNKS, CHUNK)
    times = x[:, 1, :].astype(jnp.int32).reshape(NUM_WORKERS, NUM_CHUNKS, CHUNK)
    out = _embed_sum(notes, times, token_emb, pos_emb)
    return with_layout_constraint(out, Layout(major_to_minor=(0, 1, 2)))
